# baseline (speedup 1.0000x reference)
import jax
import jax.numpy as jnp
from jax import lax
from jax.experimental import pallas as pl
from jax.experimental.pallas import tpu as pltpu
from jax.experimental.pallas import tpu_sc as plsc

N = 32768
P = 64
L = 16
NC = 2
NS = 16
NW = NC * NS
TOK_W = N // NW
TB = 128
NCHUNK = TOK_W // TB
GROUPS = TB // L


def _topk_mask_body(in4, out4, in0, in1, ot0, ot1, si0, si1, so0, so1):
    cid = lax.axis_index("c")
    sid = lax.axis_index("s")
    wid = sid * NC + cid
    jbase = wid * NCHUNK

    ins = (in0, in1)
    outs = (ot0, ot1)
    isems = (si0, si1)
    osems = (so0, so1)

    lane = lax.iota(jnp.int32, L)
    zeros_i = jnp.zeros((L,), jnp.int32)
    ones_i = jnp.ones((L,), jnp.int32)
    ninf = jnp.full((L,), float("-inf"), jnp.float32)

    def in_copy(k, b):
        return pltpu.make_async_copy(in4.at[:, jbase + k], ins[b], isems[b])

    def out_copy(k, b):
        return pltpu.make_async_copy(outs[b], out4.at[:, jbase + k], osems[b])

    in_copy(0, 0).start()

    def pair_body(pp, carry):
        for b in range(2):
            k = pp * 2 + b
            in_copy(k, b).wait()

            @pl.when(k + 1 < NCHUNK)
            def _():
                in_copy(k + 1, 1 - b).start()

            @pl.when(k >= 2)
            def _():
                out_copy(k - 2, b).wait()

            def group_body(g, gcarry):
                def octet_vals(i, oc):
                    m1, m2 = oc
                    for q in range(8):
                        v = ins[b][i, q, pl.ds(g * L, L)]
                        u = jnp.minimum(m1, v)
                        m1 = jnp.maximum(m1, v)
                        m2 = jnp.maximum(m2, u)
                    return m1, m2

                m1, m2 = lax.fori_loop(
                    0, P // 8, octet_vals, (ninf, ninf))

                def octet_mask(i, cnt):
                    for q in range(8):
                        v = ins[b][i, q, pl.ds(g * L, L)]
                        mi = jnp.where(v >= m2, ones_i, zeros_i)
                        outs[b][i, q, pl.ds(g * L, L)] = mi
                        cnt = cnt + mi
                    return cnt

                cnt = lax.fori_loop(0, P // 8, octet_mask, zeros_i)

                @pl.when(jnp.any(cnt != 2))
                def _():
                    def octet_track(i, oc):
                        m1, m2, a1, a2 = oc
                        p0 = i * 8
                        for q in range(8):
                            v = ins[b][i, q, pl.ds(g * L, L)]
                            pq = p0 + q
                            gt1 = v > m1
                            u = jnp.minimum(m1, v)
                            ui = jnp.where(gt1, a1, pq)
                            gt2 = gt1 | (u > m2)
                            m1 = jnp.maximum(m1, v)
                            a1 = jnp.where(gt1, pq, a1)
                            m2 = jnp.maximum(m2, u)
                            a2 = jnp.where(gt2, ui, a2)
                            outs[b][i, q, pl.ds(g * L, L)] = zeros_i
                        return m1, m2, a1, a2

                    t1, t2, a1, a2 = lax.fori_loop(
                        0, P // 8, octet_track,
                        (ninf, ninf, zeros_i, zeros_i))
                    tl = lane + g * L
                    plsc.store_scatter(
                        outs[b], [a1 >> 3, a1 & 7, tl], ones_i)
                    plsc.store_scatter(
                        outs[b], [a2 >> 3, a2 & 7, tl], ones_i)

                return gcarry

            lax.fori_loop(0, GROUPS, group_body, 0)
            out_copy(k, b).start()
        return carry

    lax.fori_loop(0, NCHUNK // 2, pair_body, 0)
    out_copy(NCHUNK - 2, 0).wait()
    out_copy(NCHUNK - 1, 1).wait()


def kernel(score):
    mesh = plsc.VectorSubcoreMesh(
        core_axis_name="c", subcore_axis_name="s",
        num_cores=NC, num_subcores=NS)
    s4 = score.T.reshape(P // 8, 8, N // TB, TB).transpose(0, 2, 1, 3)
    out4 = pl.kernel(
        _topk_mask_body,
        out_type=jax.ShapeDtypeStruct((P // 8, N // TB, 8, TB), jnp.int32),
        mesh=mesh,
        compiler_params=pltpu.CompilerParams(needs_layout_passes=False),
        scratch_types=[
            pltpu.VMEM((P // 8, 8, TB), jnp.float32),
            pltpu.VMEM((P // 8, 8, TB), jnp.float32),
            pltpu.VMEM((P // 8, 8, TB), jnp.int32),
            pltpu.VMEM((P // 8, 8, TB), jnp.int32),
            pltpu.SemaphoreType.DMA,
            pltpu.SemaphoreType.DMA,
            pltpu.SemaphoreType.DMA,
            pltpu.SemaphoreType.DMA,
        ],
    )(s4)
    return out4.transpose(0, 2, 1, 3).reshape(P, N).T


# speedup vs baseline: 1.5294x; 1.5294x over previous
import jax
import jax.numpy as jnp
from jax import lax
from jax.experimental import pallas as pl
from jax.experimental.pallas import tpu as pltpu
from jax.experimental.pallas import tpu_sc as plsc

N = 32768
P = 64
L = 16
NC = 2
NS = 16
NW = NC * NS
TOK_W = N // NW
TB = 128
NCHUNK = TOK_W // TB
GROUPS = TB // L


def _topk_mask_body(in4, out4, in0, in1, ot0, ot1, si0, si1, so0, so1):
    cid = lax.axis_index("c")
    sid = lax.axis_index("s")
    wid = sid * NC + cid
    jbase = wid * NCHUNK

    ins = (in0, in1)
    outs = (ot0, ot1)
    isems = (si0, si1)
    osems = (so0, so1)

    lane = lax.iota(jnp.int32, L)
    zeros_i = jnp.zeros((L,), jnp.int32)
    ones_i = jnp.ones((L,), jnp.int32)
    ninf = jnp.full((L,), float("-inf"), jnp.float32)

    def in_copy(k, b):
        return pltpu.make_async_copy(in4.at[:, jbase + k], ins[b], isems[b])

    def out_copy(k, b):
        return pltpu.make_async_copy(outs[b], out4.at[:, jbase + k], osems[b])

    in_copy(0, 0).start()

    def pair_body(pp, carry):
        for b in range(2):
            k = pp * 2 + b
            in_copy(k, b).wait()

            @pl.when(k + 1 < NCHUNK)
            def _():
                in_copy(k + 1, 1 - b).start()

            @pl.when(k >= 2)
            def _():
                out_copy(k - 2, b).wait()

            def group_body(g, gcarry):
                def octet_body(i, oc):
                    m1a, m2a, a1a, a2a, m1b, m2b, a1b, a2b = oc
                    pa = i * 8
                    for q in range(8):
                        vA = ins[b][i, q, pl.ds(g * L, L)]
                        gt1 = vA > m1a
                        u = jnp.minimum(m1a, vA)
                        ui = jnp.where(gt1, a1a, pa + q)
                        gt2 = gt1 | (u > m2a)
                        m1a = jnp.maximum(m1a, vA)
                        a1a = jnp.where(gt1, pa + q, a1a)
                        m2a = jnp.maximum(m2a, u)
                        a2a = jnp.where(gt2, ui, a2a)
                        outs[b][i, q, pl.ds(g * L, L)] = zeros_i

                        vB = ins[b][i + 4, q, pl.ds(g * L, L)]
                        ht1 = vB > m1b
                        w = jnp.minimum(m1b, vB)
                        wi = jnp.where(ht1, a1b, pa + q + 32)
                        ht2 = ht1 | (w > m2b)
                        m1b = jnp.maximum(m1b, vB)
                        a1b = jnp.where(ht1, pa + q + 32, a1b)
                        m2b = jnp.maximum(m2b, w)
                        a2b = jnp.where(ht2, wi, a2b)
                        outs[b][i + 4, q, pl.ds(g * L, L)] = zeros_i
                    return m1a, m2a, a1a, a2a, m1b, m2b, a1b, a2b

                m1a, m2a, a1a, a2a, m1b, m2b, a1b, a2b = lax.fori_loop(
                    0, 4, octet_body,
                    (ninf, ninf, zeros_i, zeros_i,
                     ninf, ninf, zeros_i, zeros_i))

                gtb = m1b > m1a
                i1 = jnp.where(gtb, a1b, a1a)
                l1 = jnp.where(gtb, m1a, m1b)
                li = jnp.where(gtb, a1a, a1b)
                w2 = jnp.where(gtb, m2b, m2a)
                wi2 = jnp.where(gtb, a2b, a2a)
                g2 = (l1 > w2) | ((l1 == w2) & (li < wi2))
                a1 = i1
                a2 = jnp.where(g2, li, wi2)

                tl = lane + g * L
                plsc.store_scatter(
                    outs[b], [a1 >> 3, a1 & 7, tl], ones_i)
                plsc.store_scatter(
                    outs[b], [a2 >> 3, a2 & 7, tl], ones_i)
                return gcarry

            lax.fori_loop(0, GROUPS, group_body, 0)
            out_copy(k, b).start()
        return carry

    lax.fori_loop(0, NCHUNK // 2, pair_body, 0)
    out_copy(NCHUNK - 2, 0).wait()
    out_copy(NCHUNK - 1, 1).wait()


def kernel(score):
    mesh = plsc.VectorSubcoreMesh(
        core_axis_name="c", subcore_axis_name="s",
        num_cores=NC, num_subcores=NS)
    s4 = score.T.reshape(P // 8, 8, N // TB, TB).transpose(0, 2, 1, 3)
    out4 = pl.kernel(
        _topk_mask_body,
        out_type=jax.ShapeDtypeStruct((P // 8, N // TB, 8, TB), jnp.int32),
        mesh=mesh,
        compiler_params=pltpu.CompilerParams(needs_layout_passes=False),
        scratch_types=[
            pltpu.VMEM((P // 8, 8, TB), jnp.float32),
            pltpu.VMEM((P // 8, 8, TB), jnp.float32),
            pltpu.VMEM((P // 8, 8, TB), jnp.int32),
            pltpu.VMEM((P // 8, 8, TB), jnp.int32),
            pltpu.SemaphoreType.DMA,
            pltpu.SemaphoreType.DMA,
            pltpu.SemaphoreType.DMA,
            pltpu.SemaphoreType.DMA,
        ],
    )(s4)
    return out4.transpose(0, 2, 1, 3).reshape(P, N).T


# speedup vs baseline: 1.5322x; 1.0018x over previous
import jax
import jax.numpy as jnp
from jax import lax
from jax.experimental import pallas as pl
from jax.experimental.pallas import tpu as pltpu
from jax.experimental.pallas import tpu_sc as plsc

N = 32768
P = 64
L = 16
NC = 2
NS = 16
NW = NC * NS
TOK_W = N // NW
TB = 128
NCHUNK = TOK_W // TB
GROUPS = TB // L


def _topk_mask_body(in4, out4, in0, in1, ot0, ot1, si0, si1, so0, so1):
    cid = lax.axis_index("c")
    sid = lax.axis_index("s")
    wid = sid * NC + cid
    jbase = wid * NCHUNK

    ins = (in0, in1)
    outs = (ot0, ot1)
    isems = (si0, si1)
    osems = (so0, so1)

    lane = lax.iota(jnp.int32, L)
    zeros_i = jnp.zeros((L,), jnp.int32)
    ones_i = jnp.ones((L,), jnp.int32)
    ninf = jnp.full((L,), float("-inf"), jnp.float32)

    def in_copy(k, b):
        return pltpu.make_async_copy(in4.at[:, jbase + k], ins[b], isems[b])

    def out_copy(k, b):
        return pltpu.make_async_copy(outs[b], out4.at[:, jbase + k], osems[b])

    in_copy(0, 0).start()

    def pair_body(pp, carry):
        for b in range(2):
            k = pp * 2 + b
            in_copy(k, b).wait()

            @pl.when(k + 1 < NCHUNK)
            def _():
                in_copy(k + 1, 1 - b).start()

            @pl.when(k >= 2)
            def _():
                out_copy(k - 2, b).wait()

            def group_body(g, gcarry):
                def octet_body(i, oc):
                    m1a, m2a, a1a, a2a, m1b, m2b, a1b, a2b = oc
                    pa = i * 8
                    for q in range(8):
                        vA = ins[b][i, q, pl.ds(g * L, L)]
                        gt1 = vA > m1a
                        u = jnp.minimum(m1a, vA)
                        a2a = jnp.where(
                            gt1, a1a, jnp.where(u > m2a, pa + q, a2a))
                        m1a = jnp.maximum(m1a, vA)
                        a1a = jnp.where(gt1, pa + q, a1a)
                        m2a = jnp.maximum(m2a, u)
                        outs[b][i, q, pl.ds(g * L, L)] = zeros_i

                        vB = ins[b][i + 4, q, pl.ds(g * L, L)]
                        ht1 = vB > m1b
                        w = jnp.minimum(m1b, vB)
                        a2b = jnp.where(
                            ht1, a1b, jnp.where(w > m2b, pa + q + 32, a2b))
                        m1b = jnp.maximum(m1b, vB)
                        a1b = jnp.where(ht1, pa + q + 32, a1b)
                        m2b = jnp.maximum(m2b, w)
                        outs[b][i + 4, q, pl.ds(g * L, L)] = zeros_i
                    return m1a, m2a, a1a, a2a, m1b, m2b, a1b, a2b

                m1a, m2a, a1a, a2a, m1b, m2b, a1b, a2b = lax.fori_loop(
                    0, 4, octet_body,
                    (ninf, ninf, zeros_i, zeros_i,
                     ninf, ninf, zeros_i, zeros_i))

                gtb = m1b > m1a
                i1 = jnp.where(gtb, a1b, a1a)
                l1 = jnp.where(gtb, m1a, m1b)
                li = jnp.where(gtb, a1a, a1b)
                w2 = jnp.where(gtb, m2b, m2a)
                wi2 = jnp.where(gtb, a2b, a2a)
                g2 = (l1 > w2) | ((l1 == w2) & (li < wi2))
                a1 = i1
                a2 = jnp.where(g2, li, wi2)

                tl = lane + g * L
                plsc.store_scatter(
                    outs[b], [a1 >> 3, a1 & 7, tl], ones_i)
                plsc.store_scatter(
                    outs[b], [a2 >> 3, a2 & 7, tl], ones_i)
                return gcarry

            lax.fori_loop(0, GROUPS, group_body, 0)
            out_copy(k, b).start()
        return carry

    lax.fori_loop(0, NCHUNK // 2, pair_body, 0)
    out_copy(NCHUNK - 2, 0).wait()
    out_copy(NCHUNK - 1, 1).wait()


def kernel(score):
    mesh = plsc.VectorSubcoreMesh(
        core_axis_name="c", subcore_axis_name="s",
        num_cores=NC, num_subcores=NS)
    s4 = score.T.reshape(P // 8, 8, N // TB, TB).transpose(0, 2, 1, 3)
    out4 = pl.kernel(
        _topk_mask_body,
        out_type=jax.ShapeDtypeStruct((P // 8, N // TB, 8, TB), jnp.int32),
        mesh=mesh,
        compiler_params=pltpu.CompilerParams(needs_layout_passes=False),
        scratch_types=[
            pltpu.VMEM((P // 8, 8, TB), jnp.float32),
            pltpu.VMEM((P // 8, 8, TB), jnp.float32),
            pltpu.VMEM((P // 8, 8, TB), jnp.int32),
            pltpu.VMEM((P // 8, 8, TB), jnp.int32),
            pltpu.SemaphoreType.DMA,
            pltpu.SemaphoreType.DMA,
            pltpu.SemaphoreType.DMA,
            pltpu.SemaphoreType.DMA,
        ],
    )(s4)
    return out4.transpose(0, 2, 1, 3).reshape(P, N).T


# speedup vs baseline: 1.5408x; 1.0056x over previous
import jax
import jax.numpy as jnp
from jax import lax
from jax.experimental import pallas as pl
from jax.experimental.pallas import tpu as pltpu
from jax.experimental.pallas import tpu_sc as plsc

N = 32768
P = 64
L = 16
NC = 2
NS = 16
NW = NC * NS
TOK_W = N // NW
TB = 128
NCHUNK = TOK_W // TB
GROUPS = TB // L


def _topk_mask_body(in4, out4, in0, in1, ot0, ot1, si0, si1, so0, so1):
    cid = lax.axis_index("c")
    sid = lax.axis_index("s")
    wid = sid * NC + cid
    jbase = wid * NCHUNK

    ins = (in0, in1)
    outs = (ot0, ot1)
    isems = (si0, si1)
    osems = (so0, so1)

    lane = lax.iota(jnp.int32, L)
    zeros_i = jnp.zeros((L,), jnp.int32)
    ones_i = jnp.ones((L,), jnp.int32)
    ninf = jnp.full((L,), float("-inf"), jnp.float32)

    def in_copy(k, b):
        return pltpu.make_async_copy(in4.at[:, jbase + k], ins[b], isems[b])

    def out_copy(k, b):
        return pltpu.make_async_copy(outs[b], out4.at[:, jbase + k], osems[b])

    in_copy(0, 0).start()

    def pair_body(pp, carry):
        for b in range(2):
            k = pp * 2 + b
            in_copy(k, b).wait()

            @pl.when(k + 1 < NCHUNK)
            def _():
                in_copy(k + 1, 1 - b).start()

            @pl.when(k >= 2)
            def _():
                out_copy(k - 2, b).wait()

            def group_body(g, gcarry):
                def octet_body(i, oc):
                    m1, m2, a1, a2 = oc
                    p0 = i * 8
                    for q in range(8):
                        v = ins[b][i, q, pl.ds(g * L, L)]
                        pq = p0 + q
                        gt1 = v > m1
                        u = jnp.minimum(m1, v)
                        ui = jnp.where(gt1, a1, pq)
                        gt2 = gt1 | (u > m2)
                        m1 = jnp.maximum(m1, v)
                        a1 = jnp.where(gt1, pq, a1)
                        m2 = jnp.maximum(m2, u)
                        a2 = jnp.where(gt2, ui, a2)
                        outs[b][i, q, pl.ds(g * L, L)] = zeros_i
                    return m1, m2, a1, a2

                m1, m2, a1, a2 = lax.fori_loop(
                    0, P // 8, octet_body, (ninf, ninf, zeros_i, zeros_i))
                tl = lane + g * L
                plsc.store_scatter(
                    outs[b], [a1 >> 3, a1 & 7, tl], ones_i)
                plsc.store_scatter(
                    outs[b], [a2 >> 3, a2 & 7, tl], ones_i)
                return gcarry

            lax.fori_loop(0, GROUPS, group_body, 0)
            out_copy(k, b).start()
        return carry

    lax.fori_loop(0, NCHUNK // 2, pair_body, 0)
    out_copy(NCHUNK - 2, 0).wait()
    out_copy(NCHUNK - 1, 1).wait()


def kernel(score):
    mesh = plsc.VectorSubcoreMesh(
        core_axis_name="c", subcore_axis_name="s",
        num_cores=NC, num_subcores=NS)
    s4 = score.T.reshape(P // 8, 8, N // TB, TB).transpose(0, 2, 1, 3)
    out4 = pl.kernel(
        _topk_mask_body,
        out_type=jax.ShapeDtypeStruct((P // 8, N // TB, 8, TB), jnp.int32),
        mesh=mesh,
        compiler_params=pltpu.CompilerParams(needs_layout_passes=False),
        scratch_types=[
            pltpu.VMEM((P // 8, 8, TB), jnp.float32),
            pltpu.VMEM((P // 8, 8, TB), jnp.float32),
            pltpu.VMEM((P // 8, 8, TB), jnp.int32),
            pltpu.VMEM((P // 8, 8, TB), jnp.int32),
            pltpu.SemaphoreType.DMA,
            pltpu.SemaphoreType.DMA,
            pltpu.SemaphoreType.DMA,
            pltpu.SemaphoreType.DMA,
        ],
    )(s4)
    return out4.transpose(0, 2, 1, 3).reshape(P, N).T
